# Initial kernel scaffold; baseline (speedup 1.0000x reference)
import jax
import jax.numpy as jnp
from jax.experimental import pallas as pl


def kernel(node_embeddings, edge_index, Wq, Wk, Wv, Wo, ln_gamma, ln_beta):
    raise NotImplementedError("write your pallas kernel here")


# speedup vs baseline: 8.9760x; 8.9760x over previous
import functools

import jax
import jax.numpy as jnp
from jax import lax
from jax.experimental import pallas as pl
from jax.experimental.pallas import tpu as pltpu
from jax.experimental.pallas import tpu_sc as plsc

N = 10000
E = 160000
D = 256
H = 8
DH = 32
QW = 64
ACCW = 80
SCALE = 1.0 / (DH ** 0.5)

NC = 2
NS = 16
C = 80
EPT = E // NS
NCHUNK = EPT // C
NPAD = 10240
ROWS_PER_TEC = NPAD // NS
ZR = 128

BLK = 512


def _qkv_body(x_ref, w_ref, out_ref):
    out_ref[0] = lax.dot_general(
        x_ref[...], w_ref[0],
        dimension_numbers=(((1,), (0,)), ((), ())),
        preferred_element_type=jnp.float32,
    )


def _qkv_proj(x2d, w12):
    grid = (pl.cdiv(N, BLK), 12)
    return pl.pallas_call(
        _qkv_body,
        grid=grid,
        in_specs=[
            pl.BlockSpec((BLK, D), lambda i, j: (i, 0)),
            pl.BlockSpec((1, D, QW), lambda i, j: (j, 0, 0)),
        ],
        out_specs=pl.BlockSpec((1, BLK, QW), lambda i, j: (j, i, 0)),
        out_shape=jax.ShapeDtypeStruct((12, N, QW), jnp.float32),
    )(x2d, w12)


def _sc_edge_attention(qcat, kcat, vcat, src, dst):
    mesh = plsc.VectorSubcoreMesh(
        core_axis_name="c", subcore_axis_name="s",
        num_cores=NC, num_subcores=NS,
    )

    @functools.partial(
        pl.kernel,
        out_type=jax.ShapeDtypeStruct((4, NPAD, ACCW), jnp.float32),
        mesh=mesh,
        scratch_types=[
            pltpu.VMEM((C,), jnp.int32),
            pltpu.VMEM((C,), jnp.int32),
            pltpu.VMEM((C,), jnp.int32),
            pltpu.VMEM((C, QW), jnp.float32),
            pltpu.VMEM((C, QW), jnp.float32),
            pltpu.VMEM((C, QW), jnp.float32),
            pltpu.VMEM((C, ACCW), jnp.float32),
            pltpu.VMEM((ZR, ACCW), jnp.float32),
            pltpu.VMEM_SHARED((NPAD, ACCW), jnp.float32),
            pltpu.SemaphoreType.DMA,
            pltpu.SemaphoreType.DMA,
            pltpu.SemaphoreType.DMA,
        ],
        compiler_params=pltpu.CompilerParams(
            needs_layout_passes=False, use_tc_tiling_on_sc=False),
    )
    def k(qcat_hbm, kcat_hbm, vcat_hbm, src_hbm, dst_hbm, out_hbm,
          srcv, qidx, kvidx, qrows, krows, vrows, contrib, zbuf,
          acc, semq, semk, semv):
        c = lax.axis_index("c")
        s = lax.axis_index("s")

        zvec = jnp.zeros((16,), jnp.float32)
        iota16 = lax.iota(jnp.int32, 16)
        row0 = s * ROWS_PER_TEC

        def zrow(r, _):
            for j in range(ACCW // 16):
                zbuf[r, pl.ds(16 * j, 16)] = zvec
            return 0
        lax.fori_loop(0, ZR, zrow, 0)

        for pas in range(2):
            quarter = c * 2 + pas
            qN = quarter * N

            for t in range(ROWS_PER_TEC // ZR):
                pltpu.sync_copy(zbuf, acc.at[pl.ds(row0 + t * ZR, ZR)])
            plsc.subcore_barrier()

            def chunk(kk, _):
                base = pl.multiple_of(s * EPT + kk * C, 8)
                pltpu.sync_copy(src_hbm.at[pl.ds(base, C)], srcv)
                pltpu.sync_copy(dst_hbm.at[pl.ds(base, C)], kvidx)
                qNv = lax.broadcast(qN, (16,))
                for i in range(C // 16):
                    sl = pl.ds(16 * i, 16)
                    qidx[sl] = srcv[sl] + qNv
                    kvidx[sl] = kvidx[sl] + qNv
                cpq = pltpu.async_copy(qcat_hbm.at[qidx], qrows, semq)
                cpk = pltpu.async_copy(kcat_hbm.at[kvidx], krows, semk)
                cpv = pltpu.async_copy(vcat_hbm.at[kvidx], vrows, semv)
                cpq.wait()
                cpk.wait()
                cpv.wait()

                def edge_body(e, _):
                    dvec = jnp.zeros((16,), jnp.float32)
                    for h in range(2):
                        sl0 = pl.ds(h * 32, 16)
                        sl1 = pl.ds(h * 32 + 16, 16)
                        a = qrows[e, sl0] * krows[e, sl0]
                        b = qrows[e, sl1] * krows[e, sl1]
                        ssum = jnp.sum(a + b)
                        pv = jnp.exp(lax.broadcast(ssum * SCALE, (16,)))
                        contrib[e, sl0] = vrows[e, sl0] * pv
                        contrib[e, sl1] = vrows[e, sl1] * pv
                        dvec = jnp.where(iota16 == h, pv, dvec)
                    contrib[e, pl.ds(64, 16)] = dvec
                    return 0
                lax.fori_loop(0, C, edge_body, 0)

                pltpu.sync_copy(contrib, acc.at[srcv], add=True)
                return 0

            lax.fori_loop(0, NCHUNK, chunk, 0)
            plsc.subcore_barrier()

            pltpu.sync_copy(acc.at[pl.ds(row0, ROWS_PER_TEC)],
                            out_hbm.at[quarter, pl.ds(row0, ROWS_PER_TEC)])
            plsc.subcore_barrier()

    return k(qcat, kcat, vcat, src, dst)


def _final_body(acc_ref, x_ref, wo_ref, g_ref, b_ref, out_ref):
    pieces = []
    for q in range(4):
        a = acc_ref[q]
        for h in range(2):
            num = a[:, h * 32:(h + 1) * 32]
            den = a[:, 64 + h:65 + h] + 1e-16
            pieces.append(num / den)
    attn = jnp.concatenate(pieces, axis=1)
    out = lax.dot_general(
        attn, wo_ref[...],
        dimension_numbers=(((1,), (1,)), ((), ())),
        preferred_element_type=jnp.float32,
    )
    y = x_ref[...] + out
    mu = jnp.mean(y, axis=-1, keepdims=True)
    d = y - mu
    var = jnp.mean(d * d, axis=-1, keepdims=True)
    out_ref[...] = d * lax.rsqrt(var + 1e-5) * g_ref[...] + b_ref[...]


def _finalize(accs, x2d, wo, gamma, beta):
    grid = (pl.cdiv(N, BLK),)
    return pl.pallas_call(
        _final_body,
        grid=grid,
        in_specs=[
            pl.BlockSpec((4, BLK, ACCW), lambda i: (0, i, 0)),
            pl.BlockSpec((BLK, D), lambda i: (i, 0)),
            pl.BlockSpec((D, D), lambda i: (0, 0)),
            pl.BlockSpec((1, D), lambda i: (0, 0)),
            pl.BlockSpec((1, D), lambda i: (0, 0)),
        ],
        out_specs=pl.BlockSpec((BLK, D), lambda i: (i, 0)),
        out_shape=jax.ShapeDtypeStruct((N, D), jnp.float32),
    )(accs, x2d, wo, gamma, beta)


def kernel(node_embeddings, edge_index, Wq, Wk, Wv, Wo, ln_gamma, ln_beta):
    x2d = node_embeddings.reshape(N, D)
    w12 = jnp.stack(
        [Wq[i * QW:(i + 1) * QW].T for i in range(4)]
        + [Wk[i * QW:(i + 1) * QW].T for i in range(4)]
        + [Wv[i * QW:(i + 1) * QW].T for i in range(4)]
    )
    qkv = _qkv_proj(x2d, w12)
    qcat = qkv[0:4].reshape(4 * N, QW)
    kcat = qkv[4:8].reshape(4 * N, QW)
    vcat = qkv[8:12].reshape(4 * N, QW)

    accs = _sc_edge_attention(qcat, kcat, vcat, edge_index[0], edge_index[1])

    out = _finalize(accs, x2d, Wo, ln_gamma.reshape(1, D), ln_beta.reshape(1, D))
    return out.reshape(1, N, D)


# speedup vs baseline: 10.0832x; 1.1234x over previous
import functools

import jax
import jax.numpy as jnp
from jax import lax
from jax.experimental import pallas as pl
from jax.experimental.pallas import tpu as pltpu
from jax.experimental.pallas import tpu_sc as plsc

N = 10000
E = 160000
D = 256
H = 8
DH = 32
QW = 64
ACCW = 80
SCALE = 1.0 / (DH ** 0.5)

NC = 2
NS = 16
C = 80
EPT = E // NS
NCHUNK = EPT // C
NPAD = 10240
ROWS_PER_TEC = NPAD // NS
ZR = 128

BLK = 512


def _qkv_body(x_ref, w_ref, out_ref):
    out_ref[0] = lax.dot_general(
        x_ref[...], w_ref[0],
        dimension_numbers=(((1,), (0,)), ((), ())),
        preferred_element_type=jnp.float32,
    )


def _qkv_proj(x2d, w12):
    grid = (pl.cdiv(N, BLK), 12)
    return pl.pallas_call(
        _qkv_body,
        grid=grid,
        in_specs=[
            pl.BlockSpec((BLK, D), lambda i, j: (i, 0)),
            pl.BlockSpec((1, D, QW), lambda i, j: (j, 0, 0)),
        ],
        out_specs=pl.BlockSpec((1, BLK, QW), lambda i, j: (j, i, 0)),
        out_shape=jax.ShapeDtypeStruct((12, N, QW), jnp.float32),
    )(x2d, w12)


def _sc_edge_attention(qcat, kcat, vcat, src, dst):
    mesh = plsc.VectorSubcoreMesh(
        core_axis_name="c", subcore_axis_name="s",
        num_cores=NC, num_subcores=NS,
    )

    @functools.partial(
        pl.kernel,
        out_type=jax.ShapeDtypeStruct((4, NPAD, ACCW), jnp.float32),
        mesh=mesh,
        scratch_types=[
            pltpu.VMEM((2, C), jnp.int32),
            pltpu.VMEM((2, C), jnp.int32),
            pltpu.VMEM((2, C), jnp.int32),
            pltpu.VMEM((2, C, QW), jnp.float32),
            pltpu.VMEM((2, C, QW), jnp.float32),
            pltpu.VMEM((2, C, QW), jnp.float32),
            pltpu.VMEM((C, ACCW), jnp.float32),
            pltpu.VMEM((ZR, ACCW), jnp.float32),
            pltpu.VMEM_SHARED((NPAD, ACCW), jnp.float32),
            pltpu.SemaphoreType.DMA,
            pltpu.SemaphoreType.DMA,
        ],
        compiler_params=pltpu.CompilerParams(
            needs_layout_passes=False, use_tc_tiling_on_sc=False),
    )
    def k(qcat_hbm, kcat_hbm, vcat_hbm, src_hbm, dst_hbm, out_hbm,
          srcv, qidx, kvidx, qrows, krows, vrows, contrib, zbuf,
          acc, sem0, sem1):
        c = lax.axis_index("c")
        s = lax.axis_index("s")

        zvec = jnp.zeros((16,), jnp.float32)
        iota16 = lax.iota(jnp.int32, 16)
        row0 = s * ROWS_PER_TEC

        def zrow(r, _):
            for j in range(ACCW // 16):
                zbuf[r, pl.ds(16 * j, 16)] = zvec
            return 0
        lax.fori_loop(0, ZR, zrow, 0)

        for pas in range(2):
            quarter = c * 2 + pas
            qN = quarter * N

            for t in range(ROWS_PER_TEC // ZR):
                pltpu.sync_copy(zbuf, acc.at[pl.ds(row0 + t * ZR, ZR)])
            plsc.subcore_barrier()

            sems = (sem0, sem1)

            def issue(kk, b):
                base = pl.multiple_of(s * EPT + kk * C, 8)
                pltpu.sync_copy(src_hbm.at[pl.ds(base, C)], srcv.at[b])
                pltpu.sync_copy(dst_hbm.at[pl.ds(base, C)], kvidx.at[b])
                qNv = lax.broadcast(qN, (16,))
                for i in range(C // 16):
                    sl = pl.ds(16 * i, 16)
                    qidx[b, sl] = srcv[b, sl] + qNv
                    kvidx[b, sl] = kvidx[b, sl] + qNv
                pltpu.async_copy(qcat_hbm.at[qidx.at[b]], qrows.at[b], sems[b])
                pltpu.async_copy(kcat_hbm.at[kvidx.at[b]], krows.at[b], sems[b])
                pltpu.async_copy(vcat_hbm.at[kvidx.at[b]], vrows.at[b], sems[b])

            def consume(b):
                pltpu.make_async_copy(
                    qcat_hbm.at[qidx.at[b]], qrows.at[b], sems[b]).wait()
                pltpu.make_async_copy(
                    kcat_hbm.at[kvidx.at[b]], krows.at[b], sems[b]).wait()
                pltpu.make_async_copy(
                    vcat_hbm.at[kvidx.at[b]], vrows.at[b], sems[b]).wait()

                def edge_body(e, _):
                    dvec = jnp.zeros((16,), jnp.float32)
                    for h in range(2):
                        sl0 = pl.ds(h * 32, 16)
                        sl1 = pl.ds(h * 32 + 16, 16)
                        a = qrows[b, e, sl0] * krows[b, e, sl0]
                        bb = qrows[b, e, sl1] * krows[b, e, sl1]
                        ssum = jnp.sum(a + bb)
                        pv = jnp.exp(lax.broadcast(ssum * SCALE, (16,)))
                        contrib[e, sl0] = vrows[b, e, sl0] * pv
                        contrib[e, sl1] = vrows[b, e, sl1] * pv
                        dvec = jnp.where(iota16 == h, pv, dvec)
                    contrib[e, pl.ds(64, 16)] = dvec
                    return 0
                lax.fori_loop(0, C, edge_body, 0)

                pltpu.sync_copy(contrib, acc.at[srcv.at[b]], add=True)

            issue(0, 0)

            def pair(i, _):
                issue(2 * i + 1, 1)
                consume(0)
                issue(2 * i + 2, 0)
                consume(1)
                return 0

            lax.fori_loop(0, (NCHUNK - 1) // 2, pair, 0)
            consume(0)
            plsc.subcore_barrier()

            pltpu.sync_copy(acc.at[pl.ds(row0, ROWS_PER_TEC)],
                            out_hbm.at[quarter, pl.ds(row0, ROWS_PER_TEC)])
            plsc.subcore_barrier()

    return k(qcat, kcat, vcat, src, dst)


def _final_body(acc_ref, x_ref, wo_ref, g_ref, b_ref, out_ref):
    pieces = []
    for q in range(4):
        a = acc_ref[q]
        for h in range(2):
            num = a[:, h * 32:(h + 1) * 32]
            den = a[:, 64 + h:65 + h] + 1e-16
            pieces.append(num / den)
    attn = jnp.concatenate(pieces, axis=1)
    out = lax.dot_general(
        attn, wo_ref[...],
        dimension_numbers=(((1,), (1,)), ((), ())),
        preferred_element_type=jnp.float32,
    )
    y = x_ref[...] + out
    mu = jnp.mean(y, axis=-1, keepdims=True)
    d = y - mu
    var = jnp.mean(d * d, axis=-1, keepdims=True)
    out_ref[...] = d * lax.rsqrt(var + 1e-5) * g_ref[...] + b_ref[...]


def _finalize(accs, x2d, wo, gamma, beta):
    grid = (pl.cdiv(N, BLK),)
    return pl.pallas_call(
        _final_body,
        grid=grid,
        in_specs=[
            pl.BlockSpec((4, BLK, ACCW), lambda i: (0, i, 0)),
            pl.BlockSpec((BLK, D), lambda i: (i, 0)),
            pl.BlockSpec((D, D), lambda i: (0, 0)),
            pl.BlockSpec((1, D), lambda i: (0, 0)),
            pl.BlockSpec((1, D), lambda i: (0, 0)),
        ],
        out_specs=pl.BlockSpec((BLK, D), lambda i: (i, 0)),
        out_shape=jax.ShapeDtypeStruct((N, D), jnp.float32),
    )(accs, x2d, wo, gamma, beta)


def kernel(node_embeddings, edge_index, Wq, Wk, Wv, Wo, ln_gamma, ln_beta):
    x2d = node_embeddings.reshape(N, D)
    w12 = jnp.stack(
        [Wq[i * QW:(i + 1) * QW].T for i in range(4)]
        + [Wk[i * QW:(i + 1) * QW].T for i in range(4)]
        + [Wv[i * QW:(i + 1) * QW].T for i in range(4)]
    )
    qkv = _qkv_proj(x2d, w12)
    qcat = qkv[0:4].reshape(4 * N, QW)
    kcat = qkv[4:8].reshape(4 * N, QW)
    vcat = qkv[8:12].reshape(4 * N, QW)

    accs = _sc_edge_attention(qcat, kcat, vcat, edge_index[0], edge_index[1])

    out = _finalize(accs, x2d, Wo, ln_gamma.reshape(1, D), ln_beta.reshape(1, D))
    return out.reshape(1, N, D)


# speedup vs baseline: 27.3158x; 2.7090x over previous
import functools

import jax
import jax.numpy as jnp
from jax import lax
from jax.experimental import pallas as pl
from jax.experimental.pallas import tpu as pltpu
from jax.experimental.pallas import tpu_sc as plsc

N = 10000
E = 160000
D = 256
H = 8
DH = 32
QW = 64
ACCW = 80
SCALE = 1.0 / (DH ** 0.5)

NC = 2
NS = 16
C = 80
EPT = E // NS
NCHUNK = EPT // C
NPAD = 10240
ROWS_PER_TEC = NPAD // NS
ZR = 128

BLK = 512


def _qkv_body(x_ref, w_ref, out_ref):
    out_ref[0] = lax.dot_general(
        x_ref[...], w_ref[0],
        dimension_numbers=(((1,), (0,)), ((), ())),
        preferred_element_type=jnp.float32,
    )


def _qkv_proj(x2d, w12):
    grid = (pl.cdiv(N, BLK), 12)
    return pl.pallas_call(
        _qkv_body,
        grid=grid,
        in_specs=[
            pl.BlockSpec((BLK, D), lambda i, j: (i, 0)),
            pl.BlockSpec((1, D, QW), lambda i, j: (j, 0, 0)),
        ],
        out_specs=pl.BlockSpec((1, BLK, QW), lambda i, j: (j, i, 0)),
        out_shape=jax.ShapeDtypeStruct((12, N, QW), jnp.float32),
    )(x2d, w12)


def _sc_edge_attention(qcat, kcat, vcat, src, dst):
    mesh = plsc.VectorSubcoreMesh(
        core_axis_name="c", subcore_axis_name="s",
        num_cores=NC, num_subcores=NS,
    )

    @functools.partial(
        pl.kernel,
        out_type=jax.ShapeDtypeStruct((4, NPAD, ACCW), jnp.float32),
        mesh=mesh,
        scratch_types=[
            pltpu.VMEM((2, C), jnp.int32),
            pltpu.VMEM((2, C), jnp.int32),
            pltpu.VMEM((2, C), jnp.int32),
            pltpu.VMEM((2, C, QW), jnp.float32),
            pltpu.VMEM((2, C, QW), jnp.float32),
            pltpu.VMEM((2, C, QW), jnp.float32),
            pltpu.VMEM((C, ACCW), jnp.float32),
            pltpu.VMEM((ZR, ACCW), jnp.float32),
            pltpu.VMEM_SHARED((NPAD, ACCW), jnp.float32),
            pltpu.SemaphoreType.DMA,
            pltpu.SemaphoreType.DMA,
        ],
        compiler_params=pltpu.CompilerParams(
            needs_layout_passes=False, use_tc_tiling_on_sc=False),
    )
    def k(qcat_hbm, kcat_hbm, vcat_hbm, src_hbm, dst_hbm, out_hbm,
          srcv, qidx, kvidx, qrows, krows, vrows, contrib, zbuf,
          acc, sem0, sem1):
        c = lax.axis_index("c")
        s = lax.axis_index("s")

        zvec = jnp.zeros((16,), jnp.float32)
        iota16 = lax.iota(jnp.int32, 16)
        row0 = s * ROWS_PER_TEC

        def zrow(r, _):
            for j in range(ACCW // 16):
                zbuf[r, pl.ds(16 * j, 16)] = zvec
            return 0
        lax.fori_loop(0, ZR, zrow, 0)

        for pas in range(2):
            quarter = c * 2 + pas
            qN = quarter * N

            for t in range(ROWS_PER_TEC // ZR):
                pltpu.sync_copy(zbuf, acc.at[pl.ds(row0 + t * ZR, ZR)])
            plsc.subcore_barrier()

            sems = (sem0, sem1)

            def issue(kk, b):
                base = pl.multiple_of(s * EPT + kk * C, 8)
                pltpu.sync_copy(src_hbm.at[pl.ds(base, C)], srcv.at[b])
                pltpu.sync_copy(dst_hbm.at[pl.ds(base, C)], kvidx.at[b])
                qNv = lax.broadcast(qN, (16,))
                for i in range(C // 16):
                    sl = pl.ds(16 * i, 16)
                    qidx[b, sl] = srcv[b, sl] + qNv
                    kvidx[b, sl] = kvidx[b, sl] + qNv
                pltpu.async_copy(qcat_hbm.at[qidx.at[b]], qrows.at[b], sems[b])
                pltpu.async_copy(kcat_hbm.at[kvidx.at[b]], krows.at[b], sems[b])
                pltpu.async_copy(vcat_hbm.at[kvidx.at[b]], vrows.at[b], sems[b])

            def consume(b):
                pltpu.make_async_copy(
                    qcat_hbm.at[qidx.at[b]], qrows.at[b], sems[b]).wait()
                pltpu.make_async_copy(
                    kcat_hbm.at[kvidx.at[b]], krows.at[b], sems[b]).wait()
                pltpu.make_async_copy(
                    vcat_hbm.at[kvidx.at[b]], vrows.at[b], sems[b]).wait()

                @plsc.parallel_loop(0, C, unroll=4)
                def edge_body(e):
                    dvec = jnp.zeros((16,), jnp.float32)
                    for h in range(2):
                        sl0 = pl.ds(h * 32, 16)
                        sl1 = pl.ds(h * 32 + 16, 16)
                        a = qrows[b, e, sl0] * krows[b, e, sl0]
                        bb = qrows[b, e, sl1] * krows[b, e, sl1]
                        ssum = jnp.sum(a + bb)
                        pv = jnp.exp(lax.broadcast(ssum * SCALE, (16,)))
                        contrib[e, sl0] = vrows[b, e, sl0] * pv
                        contrib[e, sl1] = vrows[b, e, sl1] * pv
                        dvec = jnp.where(iota16 == h, pv, dvec)
                    contrib[e, pl.ds(64, 16)] = dvec

                pltpu.sync_copy(contrib, acc.at[srcv.at[b]], add=True)

            issue(0, 0)

            def pair(i, _):
                issue(2 * i + 1, 1)
                consume(0)
                issue(2 * i + 2, 0)
                consume(1)
                return 0

            lax.fori_loop(0, (NCHUNK - 1) // 2, pair, 0)
            consume(0)
            plsc.subcore_barrier()

            pltpu.sync_copy(acc.at[pl.ds(row0, ROWS_PER_TEC)],
                            out_hbm.at[quarter, pl.ds(row0, ROWS_PER_TEC)])
            plsc.subcore_barrier()

    return k(qcat, kcat, vcat, src, dst)


def _final_body(acc_ref, x_ref, wo_ref, g_ref, b_ref, out_ref):
    pieces = []
    for q in range(4):
        a = acc_ref[q]
        for h in range(2):
            num = a[:, h * 32:(h + 1) * 32]
            den = a[:, 64 + h:65 + h] + 1e-16
            pieces.append(num / den)
    attn = jnp.concatenate(pieces, axis=1)
    out = lax.dot_general(
        attn, wo_ref[...],
        dimension_numbers=(((1,), (1,)), ((), ())),
        preferred_element_type=jnp.float32,
    )
    y = x_ref[...] + out
    mu = jnp.mean(y, axis=-1, keepdims=True)
    d = y - mu
    var = jnp.mean(d * d, axis=-1, keepdims=True)
    out_ref[...] = d * lax.rsqrt(var + 1e-5) * g_ref[...] + b_ref[...]


def _finalize(accs, x2d, wo, gamma, beta):
    grid = (pl.cdiv(N, BLK),)
    return pl.pallas_call(
        _final_body,
        grid=grid,
        in_specs=[
            pl.BlockSpec((4, BLK, ACCW), lambda i: (0, i, 0)),
            pl.BlockSpec((BLK, D), lambda i: (i, 0)),
            pl.BlockSpec((D, D), lambda i: (0, 0)),
            pl.BlockSpec((1, D), lambda i: (0, 0)),
            pl.BlockSpec((1, D), lambda i: (0, 0)),
        ],
        out_specs=pl.BlockSpec((BLK, D), lambda i: (i, 0)),
        out_shape=jax.ShapeDtypeStruct((N, D), jnp.float32),
    )(accs, x2d, wo, gamma, beta)


def kernel(node_embeddings, edge_index, Wq, Wk, Wv, Wo, ln_gamma, ln_beta):
    x2d = node_embeddings.reshape(N, D)
    w12 = jnp.stack(
        [Wq[i * QW:(i + 1) * QW].T for i in range(4)]
        + [Wk[i * QW:(i + 1) * QW].T for i in range(4)]
        + [Wv[i * QW:(i + 1) * QW].T for i in range(4)]
    )
    qkv = _qkv_proj(x2d, w12)
    qcat = qkv[0:4].reshape(4 * N, QW)
    kcat = qkv[4:8].reshape(4 * N, QW)
    vcat = qkv[8:12].reshape(4 * N, QW)

    accs = _sc_edge_attention(qcat, kcat, vcat, edge_index[0], edge_index[1])

    out = _finalize(accs, x2d, Wo, ln_gamma.reshape(1, D), ln_beta.reshape(1, D))
    return out.reshape(1, N, D)


# speedup vs baseline: 27.5030x; 1.0069x over previous
import functools

import jax
import jax.numpy as jnp
from jax import lax
from jax.experimental import pallas as pl
from jax.experimental.pallas import tpu as pltpu
from jax.experimental.pallas import tpu_sc as plsc

N = 10000
E = 160000
D = 256
H = 8
DH = 32
QW = 64
ACCW = 80
SCALE = 1.0 / (DH ** 0.5)

NC = 2
NS = 16
C = 80
EPT = E // NS
NCHUNK = EPT // C
NPAD = 10240
ROWS_PER_TEC = NPAD // NS
ZR = 128

BLK = 512


def _qkv_body(x_ref, w_ref, out_ref):
    out_ref[0] = lax.dot_general(
        x_ref[...], w_ref[0],
        dimension_numbers=(((1,), (0,)), ((), ())),
        preferred_element_type=jnp.float32,
    )


def _qkv_proj(x2d, w12):
    grid = (pl.cdiv(N, BLK), 12)
    return pl.pallas_call(
        _qkv_body,
        grid=grid,
        in_specs=[
            pl.BlockSpec((BLK, D), lambda i, j: (i, 0)),
            pl.BlockSpec((1, D, QW), lambda i, j: (j, 0, 0)),
        ],
        out_specs=pl.BlockSpec((1, BLK, QW), lambda i, j: (j, i, 0)),
        out_shape=jax.ShapeDtypeStruct((12, N, QW), jnp.float32),
    )(x2d, w12)


def _sc_edge_attention(qcat, kcat, vcat, src, dst):
    mesh = plsc.VectorSubcoreMesh(
        core_axis_name="c", subcore_axis_name="s",
        num_cores=NC, num_subcores=NS,
    )

    @functools.partial(
        pl.kernel,
        out_type=jax.ShapeDtypeStruct((4, NPAD, ACCW), jnp.float32),
        mesh=mesh,
        scratch_types=[
            pltpu.VMEM((2, C), jnp.int32),
            pltpu.VMEM((2, C), jnp.int32),
            pltpu.VMEM((2, C), jnp.int32),
            pltpu.VMEM((2, C, QW), jnp.float32),
            pltpu.VMEM((2, C, QW), jnp.float32),
            pltpu.VMEM((2, C, QW), jnp.float32),
            pltpu.VMEM((C, ACCW), jnp.float32),
            pltpu.VMEM((ZR, ACCW), jnp.float32),
            pltpu.VMEM_SHARED((NPAD, ACCW), jnp.float32),
            pltpu.SemaphoreType.DMA,
            pltpu.SemaphoreType.DMA,
        ],
        compiler_params=pltpu.CompilerParams(
            needs_layout_passes=False, use_tc_tiling_on_sc=False),
    )
    def k(qcat_hbm, kcat_hbm, vcat_hbm, src_hbm, dst_hbm, out_hbm,
          srcv, qidx, kvidx, qrows, krows, vrows, contrib, zbuf,
          acc, sem0, sem1):
        c = lax.axis_index("c")
        s = lax.axis_index("s")

        zvec = jnp.zeros((16,), jnp.float32)
        iota16 = lax.iota(jnp.int32, 16)
        row0 = s * ROWS_PER_TEC

        def zrow(r, _):
            for j in range(ACCW // 16):
                zbuf[r, pl.ds(16 * j, 16)] = zvec
            return 0
        lax.fori_loop(0, ZR, zrow, 0)

        for pas in range(2):
            quarter = c * 2 + pas
            qN = quarter * N

            for t in range(ROWS_PER_TEC // ZR):
                pltpu.sync_copy(zbuf, acc.at[pl.ds(row0 + t * ZR, ZR)])
            plsc.subcore_barrier()

            sems = (sem0, sem1)

            def issue(kk, b):
                base = pl.multiple_of(s * EPT + kk * C, 8)
                pltpu.sync_copy(src_hbm.at[pl.ds(base, C)], srcv.at[b])
                pltpu.sync_copy(dst_hbm.at[pl.ds(base, C)], kvidx.at[b])
                qNv = lax.broadcast(qN, (16,))
                for i in range(C // 16):
                    sl = pl.ds(16 * i, 16)
                    qidx[b, sl] = srcv[b, sl] + qNv
                    kvidx[b, sl] = kvidx[b, sl] + qNv
                pltpu.async_copy(qcat_hbm.at[qidx.at[b]], qrows.at[b], sems[b])
                pltpu.async_copy(kcat_hbm.at[kvidx.at[b]], krows.at[b], sems[b])
                pltpu.async_copy(vcat_hbm.at[kvidx.at[b]], vrows.at[b], sems[b])

            def consume(b):
                pltpu.make_async_copy(
                    qcat_hbm.at[qidx.at[b]], qrows.at[b], sems[b]).wait()
                pltpu.make_async_copy(
                    kcat_hbm.at[kvidx.at[b]], krows.at[b], sems[b]).wait()
                pltpu.make_async_copy(
                    vcat_hbm.at[kvidx.at[b]], vrows.at[b], sems[b]).wait()

                @plsc.parallel_loop(0, C, unroll=8)
                def edge_body(e):
                    dvec = jnp.zeros((16,), jnp.float32)
                    for h in range(2):
                        sl0 = pl.ds(h * 32, 16)
                        sl1 = pl.ds(h * 32 + 16, 16)
                        a = qrows[b, e, sl0] * krows[b, e, sl0]
                        bb = qrows[b, e, sl1] * krows[b, e, sl1]
                        ssum = jnp.sum(a + bb)
                        pv = jnp.exp(lax.broadcast(ssum * SCALE, (16,)))
                        contrib[e, sl0] = vrows[b, e, sl0] * pv
                        contrib[e, sl1] = vrows[b, e, sl1] * pv
                        dvec = jnp.where(iota16 == h, pv, dvec)
                    contrib[e, pl.ds(64, 16)] = dvec

                pltpu.sync_copy(contrib, acc.at[srcv.at[b]], add=True)

            issue(0, 0)

            def pair(i, _):
                issue(2 * i + 1, 1)
                consume(0)
                issue(2 * i + 2, 0)
                consume(1)
                return 0

            lax.fori_loop(0, (NCHUNK - 1) // 2, pair, 0)
            consume(0)
            plsc.subcore_barrier()

            pltpu.sync_copy(acc.at[pl.ds(row0, ROWS_PER_TEC)],
                            out_hbm.at[quarter, pl.ds(row0, ROWS_PER_TEC)])
            plsc.subcore_barrier()

    return k(qcat, kcat, vcat, src, dst)


def _final_body(acc_ref, x_ref, wo_ref, g_ref, b_ref, out_ref):
    pieces = []
    for q in range(4):
        a = acc_ref[q]
        for h in range(2):
            num = a[:, h * 32:(h + 1) * 32]
            den = a[:, 64 + h:65 + h] + 1e-16
            pieces.append(num / den)
    attn = jnp.concatenate(pieces, axis=1)
    out = lax.dot_general(
        attn, wo_ref[...],
        dimension_numbers=(((1,), (1,)), ((), ())),
        preferred_element_type=jnp.float32,
    )
    y = x_ref[...] + out
    mu = jnp.mean(y, axis=-1, keepdims=True)
    d = y - mu
    var = jnp.mean(d * d, axis=-1, keepdims=True)
    out_ref[...] = d * lax.rsqrt(var + 1e-5) * g_ref[...] + b_ref[...]


def _finalize(accs, x2d, wo, gamma, beta):
    grid = (pl.cdiv(N, BLK),)
    return pl.pallas_call(
        _final_body,
        grid=grid,
        in_specs=[
            pl.BlockSpec((4, BLK, ACCW), lambda i: (0, i, 0)),
            pl.BlockSpec((BLK, D), lambda i: (i, 0)),
            pl.BlockSpec((D, D), lambda i: (0, 0)),
            pl.BlockSpec((1, D), lambda i: (0, 0)),
            pl.BlockSpec((1, D), lambda i: (0, 0)),
        ],
        out_specs=pl.BlockSpec((BLK, D), lambda i: (i, 0)),
        out_shape=jax.ShapeDtypeStruct((N, D), jnp.float32),
    )(accs, x2d, wo, gamma, beta)


def kernel(node_embeddings, edge_index, Wq, Wk, Wv, Wo, ln_gamma, ln_beta):
    x2d = node_embeddings.reshape(N, D)
    w12 = jnp.stack(
        [Wq[i * QW:(i + 1) * QW].T for i in range(4)]
        + [Wk[i * QW:(i + 1) * QW].T for i in range(4)]
        + [Wv[i * QW:(i + 1) * QW].T for i in range(4)]
    )
    qkv = _qkv_proj(x2d, w12)
    qcat = qkv[0:4].reshape(4 * N, QW)
    kcat = qkv[4:8].reshape(4 * N, QW)
    vcat = qkv[8:12].reshape(4 * N, QW)

    accs = _sc_edge_attention(qcat, kcat, vcat, edge_index[0], edge_index[1])

    out = _finalize(accs, x2d, Wo, ln_gamma.reshape(1, D), ln_beta.reshape(1, D))
    return out.reshape(1, N, D)


# speedup vs baseline: 34.7794x; 1.2646x over previous
import functools

import jax
import jax.numpy as jnp
from jax import lax
from jax.experimental import pallas as pl
from jax.experimental.pallas import tpu as pltpu
from jax.experimental.pallas import tpu_sc as plsc

N = 10000
E = 160000
D = 256
H = 8
DH = 32
QW = 64
ACCW = 80
SCALE = 1.0 / (DH ** 0.5)

NC = 2
NS = 16
C = 80
EPT = E // NS
NCHUNK = EPT // C
NPAD = 10240
ROWS_PER_TEC = NPAD // NS
ZR = 128

BLK = 512


def _qkv_body(x_ref, w_ref, out_ref):
    out_ref[0] = lax.dot_general(
        x_ref[...], w_ref[0],
        dimension_numbers=(((1,), (0,)), ((), ())),
        preferred_element_type=jnp.float32,
    )


def _qkv_proj(x2d, w12):
    grid = (pl.cdiv(N, BLK), 12)
    return pl.pallas_call(
        _qkv_body,
        grid=grid,
        in_specs=[
            pl.BlockSpec((BLK, D), lambda i, j: (i, 0)),
            pl.BlockSpec((1, D, QW), lambda i, j: (j, 0, 0)),
        ],
        out_specs=pl.BlockSpec((1, BLK, QW), lambda i, j: (j, i, 0)),
        out_shape=jax.ShapeDtypeStruct((12, N, QW), jnp.float32),
    )(x2d, w12)


def _sc_edge_attention(qcat, kcat, vcat, src, dst):
    mesh = plsc.VectorSubcoreMesh(
        core_axis_name="c", subcore_axis_name="s",
        num_cores=NC, num_subcores=NS,
    )

    @functools.partial(
        pl.kernel,
        out_type=jax.ShapeDtypeStruct((4, NPAD, ACCW), jnp.float32),
        mesh=mesh,
        scratch_types=[
            pltpu.VMEM((2, C), jnp.int32),
            pltpu.VMEM((2, C), jnp.int32),
            pltpu.VMEM((2, C), jnp.int32),
            pltpu.VMEM((2, C, QW), jnp.float32),
            pltpu.VMEM((2, C, QW), jnp.float32),
            pltpu.VMEM((2, C, QW), jnp.float32),
            pltpu.VMEM((C, ACCW), jnp.float32),
            pltpu.VMEM((ZR, ACCW), jnp.float32),
            pltpu.VMEM((EPT,), jnp.int32),
            pltpu.VMEM((EPT,), jnp.int32),
            pltpu.VMEM_SHARED((NPAD, ACCW), jnp.float32),
            pltpu.SemaphoreType.DMA,
            pltpu.SemaphoreType.DMA,
        ],
        compiler_params=pltpu.CompilerParams(
            needs_layout_passes=False, use_tc_tiling_on_sc=False),
    )
    def k(qcat_hbm, kcat_hbm, vcat_hbm, src_hbm, dst_hbm, out_hbm,
          srcv, qidx, kvidx, qrows, krows, vrows, contrib, zbuf,
          srcall, dstall, acc, sem0, sem1):
        c = lax.axis_index("c")
        s = lax.axis_index("s")

        zvec = jnp.zeros((16,), jnp.float32)
        iota16 = lax.iota(jnp.int32, 16)
        row0 = s * ROWS_PER_TEC

        def zrow(r, _):
            for j in range(ACCW // 16):
                zbuf[r, pl.ds(16 * j, 16)] = zvec
            return 0
        lax.fori_loop(0, ZR, zrow, 0)
        ebase = pl.multiple_of(s * EPT, 8)
        pltpu.sync_copy(src_hbm.at[pl.ds(ebase, EPT)], srcall)
        pltpu.sync_copy(dst_hbm.at[pl.ds(ebase, EPT)], dstall)

        for pas in range(2):
            quarter = c * 2 + pas
            qN = quarter * N

            for t in range(ROWS_PER_TEC // ZR):
                pltpu.sync_copy(zbuf, acc.at[pl.ds(row0 + t * ZR, ZR)])
            plsc.subcore_barrier()

            sems = (sem0, sem1)

            def issue(kk, b):
                qNv = lax.broadcast(qN, (16,))
                for i in range(C // 16):
                    sl = pl.ds(16 * i, 16)
                    gsl = pl.ds(kk * C + 16 * i, 16)
                    sv = srcall[gsl]
                    srcv[b, sl] = sv
                    qidx[b, sl] = sv + qNv
                    kvidx[b, sl] = dstall[gsl] + qNv
                pltpu.async_copy(qcat_hbm.at[qidx.at[b]], qrows.at[b], sems[b])
                pltpu.async_copy(kcat_hbm.at[kvidx.at[b]], krows.at[b], sems[b])
                pltpu.async_copy(vcat_hbm.at[kvidx.at[b]], vrows.at[b], sems[b])

            def consume(b):
                pltpu.make_async_copy(
                    qcat_hbm.at[qidx.at[b]], qrows.at[b], sems[b]).wait()
                pltpu.make_async_copy(
                    kcat_hbm.at[kvidx.at[b]], krows.at[b], sems[b]).wait()
                pltpu.make_async_copy(
                    vcat_hbm.at[kvidx.at[b]], vrows.at[b], sems[b]).wait()

                @plsc.parallel_loop(0, C, unroll=8)
                def edge_body(e):
                    dvec = jnp.zeros((16,), jnp.float32)
                    for h in range(2):
                        sl0 = pl.ds(h * 32, 16)
                        sl1 = pl.ds(h * 32 + 16, 16)
                        a = qrows[b, e, sl0] * krows[b, e, sl0]
                        bb = qrows[b, e, sl1] * krows[b, e, sl1]
                        ssum = jnp.sum(a + bb)
                        pv = jnp.exp(lax.broadcast(ssum * SCALE, (16,)))
                        contrib[e, sl0] = vrows[b, e, sl0] * pv
                        contrib[e, sl1] = vrows[b, e, sl1] * pv
                        dvec = jnp.where(iota16 == h, pv, dvec)
                    contrib[e, pl.ds(64, 16)] = dvec

                pltpu.sync_copy(contrib, acc.at[srcv.at[b]], add=True)

            issue(0, 0)

            def pair(i, _):
                issue(2 * i + 1, 1)
                consume(0)
                issue(2 * i + 2, 0)
                consume(1)
                return 0

            lax.fori_loop(0, (NCHUNK - 1) // 2, pair, 0)
            consume(0)
            plsc.subcore_barrier()

            pltpu.sync_copy(acc.at[pl.ds(row0, ROWS_PER_TEC)],
                            out_hbm.at[quarter, pl.ds(row0, ROWS_PER_TEC)])
            plsc.subcore_barrier()

    return k(qcat, kcat, vcat, src, dst)


def _final_body(acc_ref, x_ref, wo_ref, g_ref, b_ref, out_ref):
    pieces = []
    for q in range(4):
        a = acc_ref[q]
        for h in range(2):
            num = a[:, h * 32:(h + 1) * 32]
            den = a[:, 64 + h:65 + h] + 1e-16
            pieces.append(num / den)
    attn = jnp.concatenate(pieces, axis=1)
    out = lax.dot_general(
        attn, wo_ref[...],
        dimension_numbers=(((1,), (1,)), ((), ())),
        preferred_element_type=jnp.float32,
    )
    y = x_ref[...] + out
    mu = jnp.mean(y, axis=-1, keepdims=True)
    d = y - mu
    var = jnp.mean(d * d, axis=-1, keepdims=True)
    out_ref[...] = d * lax.rsqrt(var + 1e-5) * g_ref[...] + b_ref[...]


def _finalize(accs, x2d, wo, gamma, beta):
    grid = (pl.cdiv(N, BLK),)
    return pl.pallas_call(
        _final_body,
        grid=grid,
        in_specs=[
            pl.BlockSpec((4, BLK, ACCW), lambda i: (0, i, 0)),
            pl.BlockSpec((BLK, D), lambda i: (i, 0)),
            pl.BlockSpec((D, D), lambda i: (0, 0)),
            pl.BlockSpec((1, D), lambda i: (0, 0)),
            pl.BlockSpec((1, D), lambda i: (0, 0)),
        ],
        out_specs=pl.BlockSpec((BLK, D), lambda i: (i, 0)),
        out_shape=jax.ShapeDtypeStruct((N, D), jnp.float32),
    )(accs, x2d, wo, gamma, beta)


def kernel(node_embeddings, edge_index, Wq, Wk, Wv, Wo, ln_gamma, ln_beta):
    x2d = node_embeddings.reshape(N, D)
    w12 = jnp.stack(
        [Wq[i * QW:(i + 1) * QW].T for i in range(4)]
        + [Wk[i * QW:(i + 1) * QW].T for i in range(4)]
        + [Wv[i * QW:(i + 1) * QW].T for i in range(4)]
    )
    qkv = _qkv_proj(x2d, w12)
    qcat = qkv[0:4].reshape(4 * N, QW)
    kcat = qkv[4:8].reshape(4 * N, QW)
    vcat = qkv[8:12].reshape(4 * N, QW)

    accs = _sc_edge_attention(qcat, kcat, vcat, edge_index[0], edge_index[1])

    out = _finalize(accs, x2d, Wo, ln_gamma.reshape(1, D), ln_beta.reshape(1, D))
    return out.reshape(1, N, D)
